# baseline (speedup 1.0000x reference)
import functools

import jax
import jax.numpy as jnp
from jax import lax
from jax.experimental import pallas as pl
from jax.experimental.pallas import tpu as pltpu
from jax.experimental.pallas import tpu_sc as plsc

B = 16384
D = 32
L = 20
ROW_W = 128
OUT_W = 3 * D + 1
G = 128


@functools.cache
def _build(nc: int, ns: int):
    nw = nc * ns
    bpw = B // nw
    ch = 64
    nchunk = bpw // ch
    rows_per_chunk = ch * L
    ng_text = rows_per_chunk // G
    ng_id = bpw // G

    mesh = plsc.VectorSubcoreMesh(core_axis_name="c", subcore_axis_name="s")
    cparams = pltpu.CompilerParams(
        needs_layout_passes=False, use_tc_tiling_on_sc=False)

    @functools.partial(
        pl.kernel,
        out_type=jax.ShapeDtypeStruct((B * D,), jnp.float32),
        mesh=mesh,
        scratch_types=[
            pltpu.VMEM((bpw * L,), jnp.int32),
            pltpu.VMEM((rows_per_chunk, D), jnp.float32),
            pltpu.VMEM((rows_per_chunk, D), jnp.float32),
            pltpu.VMEM((1, D), jnp.float32),
            pltpu.VMEM((bpw + 16,), jnp.float32),
            pltpu.VMEM((bpw + 16,), jnp.float32),
            pltpu.VMEM((ch * D,), jnp.float32),
            pltpu.VMEM((ch * D,), jnp.float32),
            pltpu.SemaphoreType.DMA,
            pltpu.SemaphoreType.DMA,
        ],
        compiler_params=cparams,
    )
    def launch_text(tok_hbm, ttab, pooled_hbm, tok_v, tr0, tr1, row0_v,
                    inv_v, coef_v, pb0, pb1, sem_g, sem_o):
        cid = lax.axis_index("c")
        sid = lax.axis_index("s")
        wid = cid * ns + sid
        base = wid * bpw

        pltpu.sync_copy(tok_hbm.at[pl.ds(base * L, bpw * L)], tok_v)
        pltpu.sync_copy(ttab.at[pl.ds(0, 1)], row0_v)

        trs = (tr0, tr1)
        pbs = (pb0, pb1)

        def fire(c):
            return [pltpu.async_copy(
                        ttab.at[tok_v.at[pl.ds((c * ng_text + k) * G, G)]],
                        trs[c % 2].at[pl.ds(k * G, G)], sem_g)
                    for k in range(ng_text)]

        gds = fire(0)

        iota16 = lax.iota(jnp.int32, 16)

        def cnt_body(k, carry):
            b0 = k * 16
            lane_b = iota16 + b0
            cnt = jnp.zeros((16,), jnp.float32)
            for j in range(L):
                flat = lane_b * L + j
                t = plsc.load_gather(tok_v, [flat])
                cnt = cnt + jnp.where(t != 0, jnp.float32(1.0), jnp.float32(0.0))
            inv_v[pl.ds(b0, 16)] = jnp.float32(1.0) / jnp.maximum(cnt, 1.0)
            coef_v[pl.ds(b0, 16)] = cnt - jnp.float32(L)
            return carry

        lax.fori_loop(0, bpw // 16, cnt_body, 0)

        r0a = row0_v[0, pl.ds(0, 16)]
        r0b = row0_v[0, pl.ds(16, 16)]

        ods = {}
        for c in range(nchunk):
            nxt = fire(c + 1) if c + 1 < nchunk else []
            for dsc in gds:
                dsc.wait()
            gds = nxt
            if c >= 2:
                ods[c - 2].wait()
            tr = trs[c % 2]
            pb = pbs[c % 2]

            def b_body(bl, carry, tr=tr, pb=pb, c=c):
                b_abs = c * ch + bl
                r = bl * L
                acc0 = jnp.zeros((16,), jnp.float32)
                acc1 = jnp.zeros((16,), jnp.float32)
                for j in range(L):
                    acc0 = acc0 + tr[r + j, pl.ds(0, 16)]
                    acc1 = acc1 + tr[r + j, pl.ds(16, 16)]
                coef = coef_v[pl.ds(b_abs, 16)][0]
                inv = inv_v[pl.ds(b_abs, 16)][0]
                off = bl * D
                pb[pl.ds(off, 16)] = (acc0 + coef * r0a) * inv
                pb[pl.ds(off + 16, 16)] = (acc1 + coef * r0b) * inv
                return carry

            lax.fori_loop(0, ch, b_body, 0)
            ods[c] = pltpu.async_copy(
                pb, pooled_hbm.at[pl.ds((base + c * ch) * D, ch * D)], sem_o)

        for c in range(max(0, nchunk - 2), nchunk):
            ods[c].wait()

    @functools.partial(
        pl.kernel,
        out_type=jax.ShapeDtypeStruct((B, D), jnp.float32),
        mesh=mesh,
        scratch_types=[
            pltpu.VMEM((bpw,), jnp.int32),
            pltpu.VMEM((bpw, D), jnp.float32),
            pltpu.SemaphoreType.DMA,
        ],
        compiler_params=cparams,
    )
    def launch_gu(uid_hbm, utab, urows_hbm, uid_v, u_rows, sem_g):
        cid = lax.axis_index("c")
        sid = lax.axis_index("s")
        wid = cid * ns + sid
        base = wid * bpw

        pltpu.sync_copy(uid_hbm.at[pl.ds(base, bpw)], uid_v)
        descs = [pltpu.async_copy(
                     utab.at[uid_v.at[pl.ds(k * G, G)]],
                     u_rows.at[pl.ds(k * G, G)], sem_g)
                 for k in range(ng_id)]
        for dsc in descs:
            dsc.wait()
        pltpu.sync_copy(u_rows, urows_hbm.at[pl.ds(base, bpw), :])

    @functools.partial(
        pl.kernel,
        out_type=jax.ShapeDtypeStruct((B, ROW_W), jnp.float32),
        mesh=mesh,
        scratch_types=[
            pltpu.VMEM((bpw,), jnp.int32),
            pltpu.VMEM((bpw + 16,), jnp.float32),
            pltpu.VMEM((bpw, D), jnp.float32),
            pltpu.VMEM((bpw, D), jnp.float32),
            pltpu.VMEM((bpw * D,), jnp.float32),
            pltpu.VMEM((2 * 16,), jnp.float32),
            pltpu.VMEM((ch, ROW_W), jnp.float32),
            pltpu.VMEM((ch, ROW_W), jnp.float32),
            pltpu.SemaphoreType.DMA,
            pltpu.SemaphoreType.DMA,
        ],
        compiler_params=cparams,
    )
    def launch_asm(aid_hbm, age_hbm, atab, urows_hbm, pooled_hbm,
                   params_hbm, out_hbm, aid_v, age_v, u_rows, a_rows,
                   pool_v, params_v, ob0, ob1, sem_g, sem_o):
        cid = lax.axis_index("c")
        sid = lax.axis_index("s")
        wid = cid * ns + sid
        base = wid * bpw

        pltpu.sync_copy(aid_hbm.at[pl.ds(base, bpw)], aid_v)

        descs = [pltpu.async_copy(
                     atab.at[aid_v.at[pl.ds(k * G, G)]],
                     a_rows.at[pl.ds(k * G, G)], sem_g)
                 for k in range(ng_id)]

        pltpu.sync_copy(age_hbm.at[pl.ds(base, bpw)], age_v.at[pl.ds(0, bpw)])
        pltpu.sync_copy(urows_hbm.at[pl.ds(base, bpw), :], u_rows)
        pltpu.sync_copy(pooled_hbm.at[pl.ds(base * D, bpw * D)], pool_v)
        pltpu.sync_copy(params_hbm, params_v)

        for dsc in descs:
            dsc.wait()

        mean_vec = params_v[pl.ds(0, 16)]
        scale_vec = params_v[pl.ds(16, 16)]

        obs = (ob0, ob1)
        ods = {}
        for c in range(nchunk):
            if c >= 2:
                ods[c - 2].wait()
            ob = obs[c % 2]

            def b_body(bl, carry, ob=ob, c=c):
                b_abs = c * ch + bl
                p = b_abs * D
                ob[bl, pl.ds(0, 16)] = u_rows[b_abs, pl.ds(0, 16)]
                ob[bl, pl.ds(16, 16)] = u_rows[b_abs, pl.ds(16, 16)]
                ob[bl, pl.ds(32, 16)] = a_rows[b_abs, pl.ds(0, 16)]
                ob[bl, pl.ds(48, 16)] = a_rows[b_abs, pl.ds(16, 16)]
                ob[bl, pl.ds(64, 16)] = pool_v[pl.ds(p, 16)]
                ob[bl, pl.ds(80, 16)] = pool_v[pl.ds(p + 16, 16)]
                agev = (age_v[pl.ds(b_abs, 16)] - mean_vec) * scale_vec
                ob[bl, pl.ds(96, 16)] = agev
                return carry

            lax.fori_loop(0, ch, b_body, 0)
            ods[c] = pltpu.async_copy(
                ob, out_hbm.at[pl.ds(base + c * ch, ch), :], sem_o)

        for c in range(max(0, nchunk - 2), nchunk):
            ods[c].wait()

    return launch_text, launch_gu, launch_asm


def kernel(user_ids, author_ids, author_tokens, age, user_table,
           author_table, text_table, age_mean, age_var):
    info = plsc.get_sparse_core_info()
    launch_text, launch_gu, launch_asm = _build(
        info.num_cores, info.num_subcores)
    pooled = launch_text(author_tokens.reshape(-1), text_table)
    u_rows = launch_gu(user_ids, user_table)
    params = jnp.concatenate([
        jnp.full((16,), age_mean, jnp.float32),
        jnp.full((16,), lax.rsqrt(age_var), jnp.float32),
    ])
    out = launch_asm(author_ids, age, author_table, u_rows, pooled, params)
    return out[:, :OUT_W]


# speedup vs baseline: 1.0009x; 1.0009x over previous
import functools

import jax
import jax.numpy as jnp
from jax import lax
from jax.experimental import pallas as pl
from jax.experimental.pallas import tpu as pltpu
from jax.experimental.pallas import tpu_sc as plsc

B = 16384
D = 32
L = 20
ROW_W = 128
OUT_W = 3 * D + 1
G = 128


@functools.cache
def _build(nc: int, ns: int):
    nw = nc * ns
    bpw = B // nw
    ch = 64
    nchunk = bpw // ch
    rows_per_chunk = ch * L
    ng_text = rows_per_chunk // G
    ng_id = bpw // G

    mesh = plsc.VectorSubcoreMesh(core_axis_name="c", subcore_axis_name="s")
    cparams = pltpu.CompilerParams(
        needs_layout_passes=False, use_tc_tiling_on_sc=False)

    @functools.partial(
        pl.kernel,
        out_type=jax.ShapeDtypeStruct((B * D,), jnp.float32),
        mesh=mesh,
        scratch_types=[
            pltpu.VMEM((bpw * L,), jnp.int32),
            pltpu.VMEM((rows_per_chunk, D), jnp.float32),
            pltpu.VMEM((rows_per_chunk, D), jnp.float32),
            pltpu.VMEM((1, D), jnp.float32),
            pltpu.VMEM((bpw + 16,), jnp.float32),
            pltpu.VMEM((bpw + 16,), jnp.float32),
            pltpu.VMEM((ch * D,), jnp.float32),
            pltpu.VMEM((ch * D,), jnp.float32),
            pltpu.SemaphoreType.DMA,
            pltpu.SemaphoreType.DMA,
        ],
        compiler_params=cparams,
    )
    def launch_text(tok_hbm, ttab, pooled_hbm, tok_v, tr0, tr1, row0_v,
                    inv_v, coef_v, pb0, pb1, sem_g, sem_o):
        cid = lax.axis_index("c")
        sid = lax.axis_index("s")
        wid = cid * ns + sid
        base = wid * bpw

        pltpu.sync_copy(tok_hbm.at[pl.ds(base * L, bpw * L)], tok_v)

        trs = (tr0, tr1)
        pbs = (pb0, pb1)

        def fire(c):
            return [pltpu.async_copy(
                        ttab.at[tok_v.at[pl.ds((c * ng_text + k) * G, G)]],
                        trs[c % 2].at[pl.ds(k * G, G)], sem_g)
                    for k in range(ng_text)]

        gds = fire(0)

        pltpu.sync_copy(ttab.at[pl.ds(0, 1)], row0_v)

        iota16 = lax.iota(jnp.int32, 16)

        def cnt_body(k, carry):
            b0 = k * 16
            lane_b = iota16 + b0
            cnt = jnp.zeros((16,), jnp.float32)
            for j in range(L):
                flat = lane_b * L + j
                t = plsc.load_gather(tok_v, [flat])
                cnt = cnt + jnp.where(t != 0, jnp.float32(1.0), jnp.float32(0.0))
            inv_v[pl.ds(b0, 16)] = jnp.float32(1.0) / jnp.maximum(cnt, 1.0)
            coef_v[pl.ds(b0, 16)] = cnt - jnp.float32(L)
            return carry

        lax.fori_loop(0, bpw // 16, cnt_body, 0)

        r0a = row0_v[0, pl.ds(0, 16)]
        r0b = row0_v[0, pl.ds(16, 16)]

        ods = {}
        for c in range(nchunk):
            nxt = fire(c + 1) if c + 1 < nchunk else []
            for dsc in gds:
                dsc.wait()
            gds = nxt
            if c >= 2:
                ods[c - 2].wait()
            tr = trs[c % 2]
            pb = pbs[c % 2]

            def b_body(q, carry, tr=tr, pb=pb, c=c):
                for u in range(2):
                    bl = q * 2 + u
                    b_abs = c * ch + bl
                    r = bl * L
                    acc0 = jnp.zeros((16,), jnp.float32)
                    acc1 = jnp.zeros((16,), jnp.float32)
                    for j in range(L):
                        acc0 = acc0 + tr[r + j, pl.ds(0, 16)]
                        acc1 = acc1 + tr[r + j, pl.ds(16, 16)]
                    coef = coef_v[pl.ds(b_abs, 16)][0]
                    inv = inv_v[pl.ds(b_abs, 16)][0]
                    off = bl * D
                    pb[pl.ds(off, 16)] = (acc0 + coef * r0a) * inv
                    pb[pl.ds(off + 16, 16)] = (acc1 + coef * r0b) * inv
                return carry

            lax.fori_loop(0, ch // 2, b_body, 0)
            ods[c] = pltpu.async_copy(
                pb, pooled_hbm.at[pl.ds((base + c * ch) * D, ch * D)], sem_o)

        for c in range(max(0, nchunk - 2), nchunk):
            ods[c].wait()

    @functools.partial(
        pl.kernel,
        out_type=jax.ShapeDtypeStruct((B, D), jnp.float32),
        mesh=mesh,
        scratch_types=[
            pltpu.VMEM((bpw,), jnp.int32),
            pltpu.VMEM((bpw, D), jnp.float32),
            pltpu.SemaphoreType.DMA,
        ],
        compiler_params=cparams,
    )
    def launch_gu(uid_hbm, utab, urows_hbm, uid_v, u_rows, sem_g):
        cid = lax.axis_index("c")
        sid = lax.axis_index("s")
        wid = cid * ns + sid
        base = wid * bpw

        pltpu.sync_copy(uid_hbm.at[pl.ds(base, bpw)], uid_v)
        descs = [pltpu.async_copy(
                     utab.at[uid_v.at[pl.ds(k * G, G)]],
                     u_rows.at[pl.ds(k * G, G)], sem_g)
                 for k in range(ng_id)]
        for dsc in descs:
            dsc.wait()
        pltpu.sync_copy(u_rows, urows_hbm.at[pl.ds(base, bpw), :])

    @functools.partial(
        pl.kernel,
        out_type=jax.ShapeDtypeStruct((B, ROW_W), jnp.float32),
        mesh=mesh,
        scratch_types=[
            pltpu.VMEM((bpw,), jnp.int32),
            pltpu.VMEM((bpw + 16,), jnp.float32),
            pltpu.VMEM((bpw, D), jnp.float32),
            pltpu.VMEM((bpw, D), jnp.float32),
            pltpu.VMEM((bpw * D,), jnp.float32),
            pltpu.VMEM((2 * 16,), jnp.float32),
            pltpu.VMEM((ch, ROW_W), jnp.float32),
            pltpu.VMEM((ch, ROW_W), jnp.float32),
            pltpu.SemaphoreType.DMA,
            pltpu.SemaphoreType.DMA,
        ],
        compiler_params=cparams,
    )
    def launch_asm(aid_hbm, age_hbm, atab, urows_hbm, pooled_hbm,
                   params_hbm, out_hbm, aid_v, age_v, u_rows, a_rows,
                   pool_v, params_v, ob0, ob1, sem_g, sem_o):
        cid = lax.axis_index("c")
        sid = lax.axis_index("s")
        wid = cid * ns + sid
        base = wid * bpw

        pltpu.sync_copy(aid_hbm.at[pl.ds(base, bpw)], aid_v)

        descs = [pltpu.async_copy(
                     atab.at[aid_v.at[pl.ds(k * G, G)]],
                     a_rows.at[pl.ds(k * G, G)], sem_g)
                 for k in range(ng_id)]

        descs.append(pltpu.async_copy(
            age_hbm.at[pl.ds(base, bpw)], age_v.at[pl.ds(0, bpw)], sem_g))
        descs.append(pltpu.async_copy(
            urows_hbm.at[pl.ds(base, bpw), :], u_rows, sem_g))
        descs.append(pltpu.async_copy(
            pooled_hbm.at[pl.ds(base * D, bpw * D)], pool_v, sem_g))
        descs.append(pltpu.async_copy(params_hbm, params_v, sem_g))

        for dsc in descs:
            dsc.wait()

        mean_vec = params_v[pl.ds(0, 16)]
        scale_vec = params_v[pl.ds(16, 16)]

        obs = (ob0, ob1)
        ods = {}
        for c in range(nchunk):
            if c >= 2:
                ods[c - 2].wait()
            ob = obs[c % 2]

            def b_body(q, carry, ob=ob, c=c):
                for u in range(4):
                    bl = q * 4 + u
                    b_abs = c * ch + bl
                    p = b_abs * D
                    ob[bl, pl.ds(0, 16)] = u_rows[b_abs, pl.ds(0, 16)]
                    ob[bl, pl.ds(16, 16)] = u_rows[b_abs, pl.ds(16, 16)]
                    ob[bl, pl.ds(32, 16)] = a_rows[b_abs, pl.ds(0, 16)]
                    ob[bl, pl.ds(48, 16)] = a_rows[b_abs, pl.ds(16, 16)]
                    ob[bl, pl.ds(64, 16)] = pool_v[pl.ds(p, 16)]
                    ob[bl, pl.ds(80, 16)] = pool_v[pl.ds(p + 16, 16)]
                    agev = (age_v[pl.ds(b_abs, 16)] - mean_vec) * scale_vec
                    ob[bl, pl.ds(96, 16)] = agev
                return carry

            lax.fori_loop(0, ch // 4, b_body, 0)
            ods[c] = pltpu.async_copy(
                ob, out_hbm.at[pl.ds(base + c * ch, ch), :], sem_o)

        for c in range(max(0, nchunk - 2), nchunk):
            ods[c].wait()

    return launch_text, launch_gu, launch_asm


def kernel(user_ids, author_ids, author_tokens, age, user_table,
           author_table, text_table, age_mean, age_var):
    info = plsc.get_sparse_core_info()
    launch_text, launch_gu, launch_asm = _build(
        info.num_cores, info.num_subcores)
    pooled = launch_text(author_tokens.reshape(-1), text_table)
    u_rows = launch_gu(user_ids, user_table)
    params = jnp.concatenate([
        jnp.full((16,), age_mean, jnp.float32),
        jnp.full((16,), lax.rsqrt(age_var), jnp.float32),
    ])
    out = launch_asm(author_ids, age, author_table, u_rows, pooled, params)
    return out[:, :OUT_W]


# speedup vs baseline: 1.0100x; 1.0091x over previous
import functools

import jax
import jax.numpy as jnp
from jax import lax
from jax.experimental import pallas as pl
from jax.experimental.pallas import tpu as pltpu
from jax.experimental.pallas import tpu_sc as plsc

B = 16384
D = 32
L = 20
ROW_W = 128
OUT_W = 3 * D + 1
G = 128


@functools.cache
def _build(nc: int, ns: int):
    nw = nc * ns
    bpw = B // nw
    ch = 64
    nchunk = bpw // ch
    rows_per_chunk = ch * L
    ng_text = rows_per_chunk // G
    ng_id = bpw // G

    mesh = plsc.VectorSubcoreMesh(core_axis_name="c", subcore_axis_name="s")
    cparams = pltpu.CompilerParams(
        needs_layout_passes=False, use_tc_tiling_on_sc=False)

    @functools.partial(
        pl.kernel,
        out_type=jax.ShapeDtypeStruct((B * D,), jnp.float32),
        mesh=mesh,
        scratch_types=[
            pltpu.VMEM((bpw * L,), jnp.int32),
            pltpu.VMEM((rows_per_chunk, D), jnp.float32),
            pltpu.VMEM((rows_per_chunk, D), jnp.float32),
            pltpu.VMEM((1, D), jnp.float32),
            pltpu.VMEM((bpw + 16,), jnp.float32),
            pltpu.VMEM((bpw + 16,), jnp.float32),
            pltpu.VMEM((ch * D,), jnp.float32),
            pltpu.VMEM((ch * D,), jnp.float32),
            pltpu.SemaphoreType.DMA,
            pltpu.SemaphoreType.DMA,
        ],
        compiler_params=cparams,
    )
    def launch_text(tok_hbm, ttab, pooled_hbm, tok_v, tr0, tr1, row0_v,
                    inv_v, coef_v, pb0, pb1, sem_g, sem_o):
        cid = lax.axis_index("c")
        sid = lax.axis_index("s")
        wid = cid * ns + sid
        base = wid * bpw

        pltpu.sync_copy(tok_hbm.at[pl.ds(base * L, bpw * L)], tok_v)

        trs = (tr0, tr1)
        pbs = (pb0, pb1)

        def fire(c):
            return [pltpu.async_copy(
                        ttab.at[tok_v.at[pl.ds((c * ng_text + k) * G, G)]],
                        trs[c % 2].at[pl.ds(k * G, G)], sem_g)
                    for k in range(ng_text)]

        gds = fire(0)

        pltpu.sync_copy(ttab.at[pl.ds(0, 1)], row0_v)

        iota16 = lax.iota(jnp.int32, 16)

        def cnt_body(k, carry):
            b0 = k * 16
            lane_b = iota16 + b0
            cnt = jnp.zeros((16,), jnp.float32)
            for j in range(L):
                flat = lane_b * L + j
                t = plsc.load_gather(tok_v, [flat])
                cnt = cnt + jnp.where(t != 0, jnp.float32(1.0), jnp.float32(0.0))
            inv_v[pl.ds(b0, 16)] = jnp.float32(1.0) / jnp.maximum(cnt, 1.0)
            coef_v[pl.ds(b0, 16)] = cnt - jnp.float32(L)
            return carry

        lax.fori_loop(0, bpw // 16, cnt_body, 0)

        r0a = row0_v[0, pl.ds(0, 16)]
        r0b = row0_v[0, pl.ds(16, 16)]

        ods = {}
        for c in range(nchunk):
            nxt = fire(c + 1) if c + 1 < nchunk else []
            for dsc in gds:
                dsc.wait()
            gds = nxt
            if c >= 2:
                ods[c - 2].wait()
            tr = trs[c % 2]
            pb = pbs[c % 2]

            def b_body(bl, carry, tr=tr, pb=pb, c=c):
                b_abs = c * ch + bl
                r = bl * L
                acc0 = jnp.zeros((16,), jnp.float32)
                acc1 = jnp.zeros((16,), jnp.float32)
                for j in range(L):
                    acc0 = acc0 + tr[r + j, pl.ds(0, 16)]
                    acc1 = acc1 + tr[r + j, pl.ds(16, 16)]
                coef = coef_v[pl.ds(b_abs, 16)][0]
                inv = inv_v[pl.ds(b_abs, 16)][0]
                off = bl * D
                pb[pl.ds(off, 16)] = (acc0 + coef * r0a) * inv
                pb[pl.ds(off + 16, 16)] = (acc1 + coef * r0b) * inv
                return carry

            lax.fori_loop(0, ch, b_body, 0)
            ods[c] = pltpu.async_copy(
                pb, pooled_hbm.at[pl.ds((base + c * ch) * D, ch * D)], sem_o)

        for c in range(max(0, nchunk - 2), nchunk):
            ods[c].wait()

    @functools.partial(
        pl.kernel,
        out_type=jax.ShapeDtypeStruct((B, ROW_W), jnp.float32),
        mesh=mesh,
        scratch_types=[
            pltpu.VMEM((bpw,), jnp.int32),
            pltpu.VMEM((bpw,), jnp.int32),
            pltpu.VMEM((bpw + 16,), jnp.float32),
            pltpu.VMEM((bpw, D), jnp.float32),
            pltpu.VMEM((bpw, D), jnp.float32),
            pltpu.VMEM((bpw * D,), jnp.float32),
            pltpu.VMEM((2 * 16,), jnp.float32),
            pltpu.VMEM((ch, ROW_W), jnp.float32),
            pltpu.VMEM((ch, ROW_W), jnp.float32),
            pltpu.SemaphoreType.DMA,
            pltpu.SemaphoreType.DMA,
        ],
        compiler_params=cparams,
    )
    def launch_asm(uid_hbm, aid_hbm, age_hbm, utab, atab, pooled_hbm,
                   params_hbm, out_hbm, uid_v, aid_v, age_v, u_rows, a_rows,
                   pool_v, params_v, ob0, ob1, sem_g, sem_o):
        cid = lax.axis_index("c")
        sid = lax.axis_index("s")
        wid = cid * ns + sid
        base = wid * bpw

        pltpu.sync_copy(uid_hbm.at[pl.ds(base, bpw)], uid_v)
        pltpu.sync_copy(aid_hbm.at[pl.ds(base, bpw)], aid_v)

        descs = []
        for k in range(ng_id):
            descs.append(pltpu.async_copy(
                utab.at[uid_v.at[pl.ds(k * G, G)]], u_rows.at[pl.ds(k * G, G)],
                sem_g))
            descs.append(pltpu.async_copy(
                atab.at[aid_v.at[pl.ds(k * G, G)]], a_rows.at[pl.ds(k * G, G)],
                sem_g))

        pltpu.sync_copy(age_hbm.at[pl.ds(base, bpw)], age_v.at[pl.ds(0, bpw)])
        pltpu.sync_copy(pooled_hbm.at[pl.ds(base * D, bpw * D)], pool_v)
        pltpu.sync_copy(params_hbm, params_v)

        for dsc in descs:
            dsc.wait()

        mean_vec = params_v[pl.ds(0, 16)]
        scale_vec = params_v[pl.ds(16, 16)]

        obs = (ob0, ob1)
        ods = {}
        for c in range(nchunk):
            if c >= 2:
                ods[c - 2].wait()
            ob = obs[c % 2]

            def b_body(bl, carry, ob=ob, c=c):
                b_abs = c * ch + bl
                p = b_abs * D
                ob[bl, pl.ds(0, 16)] = u_rows[b_abs, pl.ds(0, 16)]
                ob[bl, pl.ds(16, 16)] = u_rows[b_abs, pl.ds(16, 16)]
                ob[bl, pl.ds(32, 16)] = a_rows[b_abs, pl.ds(0, 16)]
                ob[bl, pl.ds(48, 16)] = a_rows[b_abs, pl.ds(16, 16)]
                ob[bl, pl.ds(64, 16)] = pool_v[pl.ds(p, 16)]
                ob[bl, pl.ds(80, 16)] = pool_v[pl.ds(p + 16, 16)]
                agev = (age_v[pl.ds(b_abs, 16)] - mean_vec) * scale_vec
                ob[bl, pl.ds(96, 16)] = agev
                return carry

            lax.fori_loop(0, ch, b_body, 0)
            ods[c] = pltpu.async_copy(
                ob, out_hbm.at[pl.ds(base + c * ch, ch), :], sem_o)

        for c in range(max(0, nchunk - 2), nchunk):
            ods[c].wait()

    return launch_text, launch_asm


def kernel(user_ids, author_ids, author_tokens, age, user_table,
           author_table, text_table, age_mean, age_var):
    info = plsc.get_sparse_core_info()
    launch_text, launch_asm = _build(info.num_cores, info.num_subcores)
    pooled = launch_text(author_tokens.reshape(-1), text_table)
    params = jnp.concatenate([
        jnp.full((16,), age_mean, jnp.float32),
        jnp.full((16,), lax.rsqrt(age_var), jnp.float32),
    ])
    out = launch_asm(user_ids, author_ids, age, user_table, author_table,
                     pooled, params)
    return out[:, :OUT_W]
